# baseline (speedup 1.0000x reference)
import functools
import math

import jax
import jax.numpy as jnp
from jax import lax
from jax.experimental import pallas as pl
from jax.experimental.pallas import tpu as pltpu
from jax.experimental.pallas import tpu_sc as plsc

B = 16384
J = 50
V = 1000000
D = 32
SCALE = math.sqrt(float(D))
NC = 2
NS = 16
NW = NC * NS
CHUNK = B // NW
LANES = 16
DPAD = D + 1

_mesh = plsc.VectorSubcoreMesh(core_axis_name="c", subcore_axis_name="s")


VSTRIP = 800
NSTRIPS = V // VSTRIP
KMAX = (NSTRIPS + NW - 1) // NW


@functools.partial(
    pl.kernel,
    mesh=_mesh,
    out_type=jax.ShapeDtypeStruct((V, D), jnp.float32),
    compiler_params=pltpu.CompilerParams(
        use_tc_tiling_on_sc=False, needs_layout_passes=False
    ),
    scratch_types=[
        pltpu.VMEM((D, VSTRIP), jnp.float32),
        pltpu.VMEM((D, VSTRIP), jnp.float32),
        pltpu.VMEM((VSTRIP, D + 1), jnp.float32),
        pltpu.SemaphoreType.DMA,
        pltpu.SemaphoreType.DMA,
    ],
)
def _tab_prep(tt_hbm, rows_hbm, a0, a1, obuf, sem0, sem1):
    wid = lax.axis_index("s") * NC + lax.axis_index("c")
    abufs = (a0, a1)
    sems = (sem0, sem1)
    lv = lax.broadcasted_iota(jnp.int32, (LANES,), 0)

    def load(k, b):
        s = wid + NW * k
        return pltpu.async_copy(
            tt_hbm.at[:, pl.ds(s * VSTRIP, VSTRIP)], abufs[b], sems[b]
        )

    def transform(a):
        def g_body(g, carry):
            tv = g * LANES + lv
            for d in range(D):
                vec = a[d, pl.ds(g * LANES, LANES)]
                plsc.store_scatter(obuf, [tv, jnp.full((LANES,), d, jnp.int32)], vec)
            return carry

        lax.fori_loop(0, VSTRIP // LANES, g_body, 0)

    load(0, 0)

    def pair_body(p, carry):
        for b in (0, 1):
            k = 2 * p + b
            s = wid + NW * k

            @pl.when(s < NSTRIPS)
            def _():
                @pl.when(s + NW < NSTRIPS)
                def _():
                    load(k + 1, 1 - b)

                pltpu.make_async_copy(
                    tt_hbm.at[:, pl.ds(s * VSTRIP, VSTRIP)], abufs[b], sems[b]
                ).wait()
                transform(abufs[b])
                pltpu.sync_copy(
                    obuf.at[:, pl.ds(0, D)], rows_hbm.at[pl.ds(s * VSTRIP, VSTRIP), :]
                )
        return carry

    lax.fori_loop(0, KMAX // 2, pair_body, 0)


@functools.partial(
    pl.kernel,
    mesh=_mesh,
    out_type=jax.ShapeDtypeStruct((J, D // 8, B // 128, 8, 128), jnp.float32),
    compiler_params=pltpu.CompilerParams(
        use_tc_tiling_on_sc=False, needs_layout_passes=False
    ),
    scratch_types=[
        pltpu.VMEM((J, CHUNK), jnp.int32),
        pltpu.VMEM((CHUNK, D), jnp.float32),
        pltpu.VMEM((CHUNK, D), jnp.float32),
        pltpu.VMEM((D // 8, CHUNK // 128, 8, 130), jnp.float32),
        pltpu.SemaphoreType.DMA,
        pltpu.SemaphoreType.DMA,
    ],
)
def _emb_lookup(tok_hbm, table_hbm, out_hbm, idx_v, rows0, rows1, tile_v, sem0, sem1):
    wid = lax.axis_index("s") * NC + lax.axis_index("c")
    b0 = wid * CHUNK
    pltpu.sync_copy(tok_hbm.at[:, pl.ds(b0, CHUNK)], idx_v)

    bufs = (rows0, rows1)
    sems = (sem0, sem1)

    def gather(j, b):
        return pltpu.async_copy(table_hbm.at[idx_v.at[j]], bufs[b], sems[b])

    lv = lax.broadcasted_iota(jnp.int32, (LANES,), 0)
    r_lo = lv >> 3
    s_all = lv & 7

    def transform(buf):
        def t_body(i4, carry):
            for u in range(4):
                i = i4 * 4 + u
                cs = jnp.full((LANES,), i >> 7, jnp.int32)
                rs = jnp.full((LANES,), i & 127, jnp.int32)
                for h in (0, 16):
                    vec = buf[i, pl.ds(h, LANES)]
                    plsc.store_scatter(
                        tile_v, [r_lo + (h // 8), cs, s_all, rs], vec * SCALE
                    )
            return carry

        lax.fori_loop(0, CHUNK // 4, t_body, 0)

    gather(0, 0)

    def pair_body(p, carry):
        for b in (0, 1):
            j = 2 * p + b

            @pl.when(j + 1 < J)
            def _():
                gather(j + 1, 1 - b)

            pltpu.make_async_copy(
                table_hbm.at[idx_v.at[j]], bufs[b], sems[b]
            ).wait()
            transform(bufs[b])
            pltpu.sync_copy(
                tile_v.at[:, :, :, pl.ds(0, 128)],
                out_hbm.at[j, :, pl.ds(wid * (CHUNK // 128), CHUNK // 128)],
            )
        return carry

    lax.fori_loop(0, J // 2, pair_body, 0)


def kernel(tokens, table):
    tok_t = tokens.T.astype(jnp.int32)
    rows = _tab_prep(table.T)
    out5 = _emb_lookup(tok_t, rows)
    return out5.transpose(2, 4, 0, 1, 3).reshape(B, J, D)


# speedup vs baseline: 4.0961x; 4.0961x over previous
import functools
import math

import jax
import jax.numpy as jnp
from jax import lax
from jax.experimental import pallas as pl
from jax.experimental.pallas import tpu as pltpu
from jax.experimental.pallas import tpu_sc as plsc

B = 16384
J = 50
V = 1000000
D = 32
SCALE = math.sqrt(float(D))
NC = 2
NS = 16
NW = NC * NS
CHUNK = B // NW
LANES = 16
DPAD = D + 1

_mesh = plsc.VectorSubcoreMesh(core_axis_name="c", subcore_axis_name="s")


@functools.partial(
    pl.kernel,
    mesh=_mesh,
    out_type=jax.ShapeDtypeStruct((J, D // 8, B // 128, 8, 128), jnp.float32),
    compiler_params=pltpu.CompilerParams(
        use_tc_tiling_on_sc=False, needs_layout_passes=False
    ),
    scratch_types=[
        pltpu.VMEM((J, CHUNK), jnp.int32),
        pltpu.VMEM((CHUNK, D), jnp.float32),
        pltpu.VMEM((CHUNK, D), jnp.float32),
        pltpu.VMEM((D // 8, CHUNK // 128, 8, 130), jnp.float32),
        pltpu.SemaphoreType.DMA,
        pltpu.SemaphoreType.DMA,
    ],
)
def _emb_lookup(tok_hbm, table_hbm, out_hbm, idx_v, rows0, rows1, tile_v, sem0, sem1):
    wid = lax.axis_index("s") * NC + lax.axis_index("c")
    b0 = wid * CHUNK
    pltpu.sync_copy(tok_hbm.at[:, pl.ds(b0, CHUNK)], idx_v)

    bufs = (rows0, rows1)
    sems = (sem0, sem1)

    def gather(j, b):
        return pltpu.async_copy(table_hbm.at[idx_v.at[j]], bufs[b], sems[b])

    lv = lax.broadcasted_iota(jnp.int32, (LANES,), 0)
    r_lo = lv >> 3
    s_all = lv & 7

    def transform(buf):
        def t_body(i4, carry):
            for u in range(4):
                i = i4 * 4 + u
                cs = jnp.full((LANES,), i >> 7, jnp.int32)
                rs = jnp.full((LANES,), i & 127, jnp.int32)
                for h in (0, 16):
                    vec = buf[i, pl.ds(h, LANES)]
                    plsc.store_scatter(
                        tile_v, [r_lo + (h // 8), cs, s_all, rs], vec * SCALE
                    )
            return carry

        lax.fori_loop(0, CHUNK // 4, t_body, 0)

    gather(0, 0)

    def pair_body(p, carry):
        for b in (0, 1):
            j = 2 * p + b

            @pl.when(j + 1 < J)
            def _():
                gather(j + 1, 1 - b)

            pltpu.make_async_copy(
                table_hbm.at[idx_v.at[j]], bufs[b], sems[b]
            ).wait()
            transform(bufs[b])
            pltpu.sync_copy(
                tile_v.at[:, :, :, pl.ds(0, 128)],
                out_hbm.at[j, :, pl.ds(wid * (CHUNK // 128), CHUNK // 128)],
            )
        return carry

    lax.fori_loop(0, J // 2, pair_body, 0)


def kernel(tokens, table):
    tok_t = tokens.T.astype(jnp.int32)
    out5 = _emb_lookup(tok_t, table)
    return out5.transpose(2, 4, 0, 1, 3).reshape(B, J, D)
